# Initial kernel scaffold; baseline (speedup 1.0000x reference)
import jax
import jax.numpy as jnp
from jax.experimental import pallas as pl


def kernel(depth_indices, node_type_indices, depth_table, type_table, W, b):
    raise NotImplementedError("write your pallas kernel here")


# speedup vs baseline: 19.8425x; 19.8425x over previous
import jax
import jax.numpy as jnp
from jax.experimental import pallas as pl
from jax.experimental.pallas import tpu as pltpu

HIDDEN = 64
MAX_DEPTH = 8
NUM_TYPES = 16
K = 32

BLK_TOK = 32768


def _body(didx_ref, tidx_ref, dtab_ref, ttab_ref, w_ref, b_ref, out_ref):
    w = w_ref[...]
    pd = jax.lax.dot_general(dtab_ref[...], w[:, :HIDDEN],
                             (((1,), (1,)), ((), ())),
                             preferred_element_type=jnp.float32) + b_ref[...]
    pt = jax.lax.dot_general(ttab_ref[...], w[:, HIDDEN:],
                             (((1,), (1,)), ((), ())),
                             preferred_element_type=jnp.float32)
    ptab = jnp.concatenate(
        [pd, pt, jnp.zeros((K - MAX_DEPTH - NUM_TYPES, HIDDEN), jnp.float32)], axis=0)

    d = jnp.broadcast_to(didx_ref[0], (K, BLK_TOK))
    c = jnp.broadcast_to(tidx_ref[0], (K, BLK_TOK))
    iota = jax.lax.broadcasted_iota(jnp.int32, (K, BLK_TOK), 0)
    two_hot_t = jnp.where((iota == d) | (iota == c + MAX_DEPTH), 1.0, 0.0)
    out_ref[...] = jax.lax.dot_general(two_hot_t, ptab,
                                       (((0,), (0,)), ((), ())),
                                       preferred_element_type=jnp.float32)


def kernel(depth_indices, node_type_indices, depth_table, type_table, W, b):
    B, L = depth_indices.shape
    n_tok = B * L
    grid = n_tok // BLK_TOK
    di = depth_indices.reshape(grid, 1, BLK_TOK)
    ci = node_type_indices.reshape(grid, 1, BLK_TOK)

    out = pl.pallas_call(
        _body,
        grid=(grid,),
        in_specs=[
            pl.BlockSpec((1, 1, BLK_TOK), lambda i: (i, 0, 0)),
            pl.BlockSpec((1, 1, BLK_TOK), lambda i: (i, 0, 0)),
            pl.BlockSpec((MAX_DEPTH, HIDDEN), lambda i: (0, 0)),
            pl.BlockSpec((NUM_TYPES, HIDDEN), lambda i: (0, 0)),
            pl.BlockSpec((HIDDEN, 2 * HIDDEN), lambda i: (0, 0)),
            pl.BlockSpec((1, HIDDEN), lambda i: (0, 0)),
        ],
        out_specs=pl.BlockSpec((BLK_TOK, HIDDEN), lambda i: (i, 0)),
        out_shape=jax.ShapeDtypeStruct((n_tok, HIDDEN), jnp.float32),
        compiler_params=pltpu.CompilerParams(
            dimension_semantics=("arbitrary",)),
    )(di, ci, depth_table, type_table, W, b.reshape(1, HIDDEN))
    return out.reshape(B, L, HIDDEN)
